# Initial kernel scaffold; baseline (speedup 1.0000x reference)
import jax
import jax.numpy as jnp
from jax.experimental import pallas as pl


def kernel(tokenized_precursor, table):
    raise NotImplementedError("write your pallas kernel here")


# speedup vs baseline: 3.2762x; 3.2762x over previous
import functools

import jax
import jax.numpy as jnp
from jax import lax
from jax.experimental import pallas as pl
from jax.experimental.pallas import tpu as pltpu
from jax.experimental.pallas import tpu_sc as plsc

D_MODEL = 128
BATCH = 4096
HIST = 50
B_TOTAL = BATCH * HIST

NUM_CORES = 2
NUM_SUBCORES = 16
NUM_WORKERS = NUM_CORES * NUM_SUBCORES
B_PER_W = B_TOTAL // NUM_WORKERS

K_SUB = 5
CHUNK = K_SUB * 128
N_CHUNKS = B_PER_W // CHUNK


def _make_gather():
    mesh = plsc.VectorSubcoreMesh(core_axis_name="c", subcore_axis_name="s")

    @functools.partial(
        pl.kernel,
        mesh=mesh,
        out_type=jax.ShapeDtypeStruct((B_TOTAL, D_MODEL), jnp.float32),
        scratch_types=[
            pltpu.VMEM((CHUNK,), jnp.int32),
            pltpu.VMEM((CHUNK, D_MODEL), jnp.float32),
            pltpu.SemaphoreType.DMA,
        ],
    )
    def gather_kernel(idx_hbm, table_hbm, out_hbm, idx_v, rows_v, sem):
        wid = lax.axis_index("s") * NUM_CORES + lax.axis_index("c")
        base = wid * B_PER_W

        def body(g, _):
            off = base + g * CHUNK
            pltpu.sync_copy(idx_hbm.at[pl.ds(off, CHUNK)], idx_v)
            copies = []
            for j in range(K_SUB):
                copies.append(
                    pltpu.async_copy(
                        table_hbm.at[idx_v.at[pl.ds(j * 128, 128)]],
                        rows_v.at[pl.ds(j * 128, 128)],
                        sem,
                    )
                )
            for c in copies:
                c.wait()
            pltpu.sync_copy(rows_v, out_hbm.at[pl.ds(off, CHUNK)])
            return 0

        lax.fori_loop(0, N_CHUNKS, body, 0)

    return gather_kernel


_gather = _make_gather()


def kernel(tokenized_precursor, table):
    idx = tokenized_precursor.reshape(B_TOTAL).astype(jnp.int32)
    out = _gather(idx, table)
    return out.reshape(BATCH, HIST, D_MODEL)


# speedup vs baseline: 3.3516x; 1.0230x over previous
import functools

import jax
import jax.numpy as jnp
from jax import lax
from jax.experimental import pallas as pl
from jax.experimental.pallas import tpu as pltpu
from jax.experimental.pallas import tpu_sc as plsc

D_MODEL = 128
BATCH = 4096
HIST = 50
B_TOTAL = BATCH * HIST

NUM_CORES = 2
NUM_SUBCORES = 16
NUM_WORKERS = NUM_CORES * NUM_SUBCORES
B_PER_W = B_TOTAL // NUM_WORKERS

CHUNK = 128
N_CHUNKS = B_PER_W // CHUNK
NBUF = 5
LAG = 2


def _make_gather():
    mesh = plsc.VectorSubcoreMesh(core_axis_name="c", subcore_axis_name="s")

    @functools.partial(
        pl.kernel,
        mesh=mesh,
        out_type=jax.ShapeDtypeStruct((B_TOTAL, D_MODEL), jnp.float32),
        scratch_types=[
            pltpu.VMEM((B_PER_W,), jnp.int32),
            pltpu.VMEM((NBUF * CHUNK, D_MODEL), jnp.float32),
            pltpu.SemaphoreType.DMA((NBUF,)),
            pltpu.SemaphoreType.DMA((NBUF,)),
        ],
    )
    def gather_kernel(idx_hbm, table_hbm, out_hbm, idx_v, rows_v, sem_g, sem_o):
        wid = lax.axis_index("s") * NUM_CORES + lax.axis_index("c")
        base = wid * B_PER_W

        def fire_gather(v, b):
            return pltpu.async_copy(
                table_hbm.at[idx_v.at[pl.ds(v * CHUNK, CHUNK)]],
                rows_v.at[pl.ds(b * CHUNK, CHUNK)],
                sem_g.at[b],
            )

        def fire_out(v, b):
            return pltpu.async_copy(
                rows_v.at[pl.ds(b * CHUNK, CHUNK)],
                out_hbm.at[pl.ds(base + v * CHUNK, CHUNK)],
                sem_o.at[b],
            )

        def wait_gather(b):
            pltpu.make_async_copy(
                table_hbm.at[idx_v.at[pl.ds(0, CHUNK)]],
                rows_v.at[pl.ds(b * CHUNK, CHUNK)],
                sem_g.at[b],
            ).wait()

        def wait_out(b):
            pltpu.make_async_copy(
                rows_v.at[pl.ds(b * CHUNK, CHUNK)],
                out_hbm.at[pl.ds(base, CHUNK)],
                sem_o.at[b],
            ).wait()

        pltpu.sync_copy(idx_hbm.at[pl.ds(base, B_PER_W)], idx_v)

        for v in range(NBUF):
            fire_gather(v, v)
            if v >= LAG:
                wait_gather(v - LAG)
                fire_out(v - LAG, v - LAG)

        def round_body(r, _):
            for b in range(NBUF):
                v = r * NBUF + b
                wait_out(b)
                fire_gather(v, b)
                bl = (b - LAG) % NBUF
                wait_gather(bl)
                fire_out(v - LAG, bl)
            return 0

        lax.fori_loop(1, N_CHUNKS // NBUF, round_body, 0)

        for v in range(N_CHUNKS - LAG, N_CHUNKS):
            b = v % NBUF
            wait_gather(b)
            fire_out(v, b)
        for b in range(NBUF):
            wait_out(b)

    return gather_kernel


_gather = _make_gather()


def kernel(tokenized_precursor, table):
    idx = tokenized_precursor.reshape(B_TOTAL).astype(jnp.int32)
    out = _gather(idx, table)
    return out.reshape(BATCH, HIST, D_MODEL)


# speedup vs baseline: 5.7745x; 1.7229x over previous
import functools

import jax
import jax.numpy as jnp
from jax import lax
from jax.experimental import pallas as pl
from jax.experimental.pallas import tpu as pltpu
from jax.experimental.pallas import tpu_sc as plsc

D_MODEL = 128
BATCH = 4096
HIST = 50

NUM_CORES = 2
NUM_SUBCORES = 16
NUM_WORKERS = NUM_CORES * NUM_SUBCORES
ROWS_PER_W = BATCH // NUM_WORKERS

NB = 8
N_CHUNKS = ROWS_PER_W // NB
NBUF = 2


def _make_gather():
    mesh = plsc.VectorSubcoreMesh(core_axis_name="c", subcore_axis_name="s")

    @functools.partial(
        pl.kernel,
        mesh=mesh,
        out_type=jax.ShapeDtypeStruct((BATCH, HIST, D_MODEL), jnp.float32),
        scratch_types=[
            pltpu.VMEM((NBUF * NB, HIST), jnp.int32),
            pltpu.VMEM((NBUF * NB, HIST, D_MODEL), jnp.float32),
            pltpu.SemaphoreType.DMA((NBUF,)),
            pltpu.SemaphoreType.DMA((NBUF,)),
        ],
    )
    def gather_kernel(idx_hbm, table_hbm, out_hbm, idx_v, rows_v, sem_g, sem_o):
        wid = lax.axis_index("s") * NUM_CORES + lax.axis_index("c")
        tbase = wid * ROWS_PER_W

        def load_idx(v, s):
            pltpu.sync_copy(
                idx_hbm.at[pl.ds(tbase + v * NB, NB)],
                idx_v.at[pl.ds(s * NB, NB)],
            )

        def fire_gathers(s):
            for b2 in range(NB):
                pltpu.async_copy(
                    table_hbm.at[idx_v.at[s * NB + b2]],
                    rows_v.at[s * NB + b2],
                    sem_g.at[s],
                )

        def drain_gathers(s):
            for b2 in range(NB):
                pltpu.make_async_copy(
                    table_hbm.at[idx_v.at[s * NB + b2]],
                    rows_v.at[s * NB + b2],
                    sem_g.at[s],
                ).wait()

        def fire_out(v, s):
            pltpu.async_copy(
                rows_v.at[pl.ds(s * NB, NB)],
                out_hbm.at[pl.ds(tbase + v * NB, NB)],
                sem_o.at[s],
            )

        def wait_out(s):
            pltpu.make_async_copy(
                rows_v.at[pl.ds(s * NB, NB)],
                out_hbm.at[pl.ds(tbase, NB)],
                sem_o.at[s],
            ).wait()

        load_idx(0, 0)
        fire_gathers(0)
        load_idx(1, 1)
        fire_gathers(1)
        drain_gathers(0)
        fire_out(0, 0)

        def round_body(r, _):
            for par in range(2):
                v = 2 * r + par
                s = par
                wait_out(s)
                load_idx(v, s)
                fire_gathers(s)
                drain_gathers(1 - s)
                fire_out(v - 1, 1 - s)
            return 0

        lax.fori_loop(1, N_CHUNKS // 2, round_body, 0)

        drain_gathers(1)
        fire_out(N_CHUNKS - 1, 1)
        wait_out(0)
        wait_out(1)

    return gather_kernel


_gather = _make_gather()


def kernel(tokenized_precursor, table):
    idx = tokenized_precursor.astype(jnp.int32)
    return _gather(idx, table)


# speedup vs baseline: 5.8508x; 1.0132x over previous
import functools

import jax
import jax.numpy as jnp
from jax import lax
from jax.experimental import pallas as pl
from jax.experimental.pallas import tpu as pltpu
from jax.experimental.pallas import tpu_sc as plsc

D_MODEL = 128
BATCH = 4096
HIST = 50

NUM_CORES = 2
NUM_SUBCORES = 16
NUM_WORKERS = NUM_CORES * NUM_SUBCORES
ROWS_PER_W = BATCH // NUM_WORKERS

NB = 8
N_CHUNKS = ROWS_PER_W // NB
NBUF = 2


def _make_gather():
    mesh = plsc.VectorSubcoreMesh(core_axis_name="c", subcore_axis_name="s")

    @functools.partial(
        pl.kernel,
        mesh=mesh,
        out_type=jax.ShapeDtypeStruct((BATCH, HIST, D_MODEL), jnp.float32),
        compiler_params=pltpu.CompilerParams(use_tc_tiling_on_sc=True),
        scratch_types=[
            pltpu.VMEM((NBUF * NB, HIST), jnp.int32),
            pltpu.VMEM((NBUF * NB, HIST, D_MODEL), jnp.float32),
            pltpu.SemaphoreType.DMA((NBUF,)),
            pltpu.SemaphoreType.DMA((NBUF,)),
        ],
    )
    def gather_kernel(idx_hbm, table_hbm, out_hbm, idx_v, rows_v, sem_g, sem_o):
        wid = lax.axis_index("s") * NUM_CORES + lax.axis_index("c")
        tbase = wid * ROWS_PER_W

        def load_idx(v, s):
            pltpu.sync_copy(
                idx_hbm.at[pl.ds(tbase + v * NB, NB)],
                idx_v.at[pl.ds(s * NB, NB)],
            )

        def fire_gathers(s):
            for b2 in range(NB):
                pltpu.async_copy(
                    table_hbm.at[idx_v.at[s * NB + b2]],
                    rows_v.at[s * NB + b2],
                    sem_g.at[s],
                )

        def drain_gathers(s):
            for b2 in range(NB):
                pltpu.make_async_copy(
                    table_hbm.at[idx_v.at[s * NB + b2]],
                    rows_v.at[s * NB + b2],
                    sem_g.at[s],
                ).wait()

        def fire_out(v, s):
            pltpu.async_copy(
                rows_v.at[pl.ds(s * NB, NB)],
                out_hbm.at[pl.ds(tbase + v * NB, NB)],
                sem_o.at[s],
            )

        def wait_out(s):
            pltpu.make_async_copy(
                rows_v.at[pl.ds(s * NB, NB)],
                out_hbm.at[pl.ds(tbase, NB)],
                sem_o.at[s],
            ).wait()

        load_idx(0, 0)
        fire_gathers(0)
        load_idx(1, 1)
        fire_gathers(1)
        drain_gathers(0)
        fire_out(0, 0)

        def round_body(r, _):
            for par in range(2):
                v = 2 * r + par
                s = par
                wait_out(s)
                load_idx(v, s)
                fire_gathers(s)
                drain_gathers(1 - s)
                fire_out(v - 1, 1 - s)
            return 0

        lax.fori_loop(1, N_CHUNKS // 2, round_body, 0)

        drain_gathers(1)
        fire_out(N_CHUNKS - 1, 1)
        wait_out(0)
        wait_out(1)

    return gather_kernel


_gather = _make_gather()


def kernel(tokenized_precursor, table):
    idx = tokenized_precursor.astype(jnp.int32)
    return _gather(idx, table)


# speedup vs baseline: 10.4171x; 1.7805x over previous
import functools

import jax
import jax.numpy as jnp
from jax import lax
from jax.experimental import pallas as pl
from jax.experimental.pallas import tpu as pltpu
from jax.experimental.pallas import tpu_sc as plsc

D_MODEL = 128
BATCH = 4096
HIST = 50
B_TOTAL = BATCH * HIST

NUM_CORES = 2
NUM_SUBCORES = 16
NUM_WORKERS = NUM_CORES * NUM_SUBCORES
B_PER_W = B_TOTAL // NUM_WORKERS

CHUNK = 128
N_CHUNKS = B_PER_W // CHUNK
NBUF = 5
LAG = 2


def _make_gather():
    mesh = plsc.VectorSubcoreMesh(core_axis_name="c", subcore_axis_name="s")

    @functools.partial(
        pl.kernel,
        mesh=mesh,
        out_type=jax.ShapeDtypeStruct((B_TOTAL, D_MODEL), jnp.float32),
        scratch_types=[
            pltpu.VMEM((B_PER_W,), jnp.int32),
            pltpu.VMEM((NBUF * CHUNK, D_MODEL), jnp.float32),
            pltpu.SemaphoreType.DMA((NBUF,)),
            pltpu.SemaphoreType.DMA((NBUF,)),
        ],
    )
    def gather_kernel(idx_hbm, table_hbm, out_hbm, idx_v, rows_v, sem_g, sem_o):
        wid = lax.axis_index("s") * NUM_CORES + lax.axis_index("c")
        base = wid * B_PER_W

        def fire_gather(v, b):
            return pltpu.async_copy(
                table_hbm.at[idx_v.at[pl.ds(v * CHUNK, CHUNK)]],
                rows_v.at[pl.ds(b * CHUNK, CHUNK)],
                sem_g.at[b],
            )

        def fire_out(v, b):
            return pltpu.async_copy(
                rows_v.at[pl.ds(b * CHUNK, CHUNK)],
                out_hbm.at[pl.ds(base + v * CHUNK, CHUNK)],
                sem_o.at[b],
            )

        def wait_gather(b):
            pltpu.make_async_copy(
                table_hbm.at[idx_v.at[pl.ds(0, CHUNK)]],
                rows_v.at[pl.ds(b * CHUNK, CHUNK)],
                sem_g.at[b],
            ).wait()

        def wait_out(b):
            pltpu.make_async_copy(
                rows_v.at[pl.ds(b * CHUNK, CHUNK)],
                out_hbm.at[pl.ds(base, CHUNK)],
                sem_o.at[b],
            ).wait()

        pltpu.sync_copy(idx_hbm.at[pl.ds(base, B_PER_W)], idx_v)

        for v in range(NBUF):
            fire_gather(v, v)
            if v >= LAG:
                wait_gather(v - LAG)
                fire_out(v - LAG, v - LAG)

        def round_body(r, _):
            for b in range(NBUF):
                v = r * NBUF + b
                wait_out(b)
                fire_gather(v, b)
                bl = (b - LAG) % NBUF
                wait_gather(bl)
                fire_out(v - LAG, bl)
            return 0

        lax.fori_loop(1, N_CHUNKS // NBUF, round_body, 0)

        for v in range(N_CHUNKS - LAG, N_CHUNKS):
            b = v % NBUF
            wait_gather(b)
            fire_out(v, b)
        for b in range(NBUF):
            wait_out(b)

    return gather_kernel


_gather = _make_gather()


def kernel(tokenized_precursor, table):
    idx = tokenized_precursor.astype(jnp.int32).T.reshape(B_TOTAL)
    out = _gather(idx, table)
    return out.reshape(HIST, BATCH, D_MODEL).transpose(1, 0, 2)
